# baseline (speedup 1.0000x reference)
import functools

import jax
import jax.numpy as jnp
from jax import lax
from jax.experimental import pallas as pl
from jax.experimental.pallas import tpu as pltpu
from jax.experimental.pallas import tpu_sc as plsc

H = 1024
HH = H // 2
I = 512
E = 8
NG = 4
T = 2048
BLK = 128
LOG_BLK = 7
NBLK = 40
NSLOT = NBLK * BLK
NTILES = 32
RCH = 128


def _pack_bf16(lo_f32, hi_f32):
    lo = lax.bitcast_convert_type(lo_f32.astype(jnp.bfloat16),
                                  jnp.uint16).astype(jnp.uint32)
    hi = lax.bitcast_convert_type(hi_f32.astype(jnp.bfloat16),
                                  jnp.uint16).astype(jnp.uint32)
    return lax.bitcast_convert_type(lo | (hi << 16), jnp.int32)


def _unpack_bf16(p):
    lo = lax.bitcast_convert_type(p << 16, jnp.float32)
    hi = lax.bitcast_convert_type(p & jnp.int32(-65536), jnp.float32)
    return lo, hi


def _routing_kernel(x_ref, gw_ref, eb_ref, w2_ref, s2_ref, be_ref, xp_ref):
    x = x_ref[...]
    gw = gw_ref[...]
    logits_t = lax.dot_general(gw, x, (((1,), (1,)), ((), ())),
                               preferred_element_type=jnp.float32)
    scores = jax.nn.sigmoid(logits_t)
    sfc = scores + eb_ref[...]
    s = [sfc[e:e + 1, :] for e in range(E)]
    sc = [scores[e:e + 1, :] for e in range(E)]
    g = [s[2 * i] + s[2 * i + 1] for i in range(NG)]
    gsel = []
    for i in range(NG):
        r = jnp.zeros_like(g[0])
        for j in range(NG):
            if j == i:
                continue
            beats = (g[j] > g[i]) | ((g[j] == g[i]) & (j < i))
            r = r + beats.astype(jnp.float32)
        gsel.append(r < 2.0)
    tmp = [jnp.where(gsel[e // 2], s[e], 0.0) for e in range(E)]
    esel = []
    for e in range(E):
        r = jnp.zeros_like(g[0])
        for e2 in range(E):
            if e2 == e:
                continue
            beats = (tmp[e2] > tmp[e]) | ((tmp[e2] == tmp[e]) & (e2 < e))
            r = r + beats.astype(jnp.float32)
        esel.append(r < 2.0)
    w = [jnp.where(esel[e], sc[e], 0.0) for e in range(E)]
    wsum = w[0]
    for e in range(1, E):
        wsum = wsum + w[e]
    inv = 1.0 / (wsum + 1e-20)
    wn = [w[e] * inv for e in range(E)]
    sel_f = [esel[e].astype(jnp.float32) for e in range(E)]
    sel_mat = jnp.concatenate(sel_f, axis=0)

    CH = 256
    iot_r = lax.broadcasted_iota(jnp.int32, (CH, CH), 0)
    iot_c = lax.broadcasted_iota(jnp.int32, (CH, CH), 1)
    tril = (iot_r < iot_c).astype(jnp.float32)
    ones_ch = jnp.ones((CH, 1), jnp.float32)
    carry = jnp.zeros((E, 1), jnp.float32)
    rank_chunks = []
    for cidx in range(T // CH):
        blk = sel_mat[:, cidx * CH:(cidx + 1) * CH]
        rank_chunks.append(
            lax.dot_general(blk, tril, (((1,), (0,)), ((), ())),
                            preferred_element_type=jnp.float32) + carry)
        carry = carry + lax.dot_general(blk, ones_ch, (((1,), (0,)), ((), ())),
                                        preferred_element_type=jnp.float32)
    rank_f = jnp.concatenate(rank_chunks, axis=1)
    counts = carry
    c_i = counts.astype(jnp.int32)
    pc = ((c_i + BLK - 1) >> LOG_BLK) << LOG_BLK
    e_r = lax.broadcasted_iota(jnp.int32, (E, E), 0)
    e_c = lax.broadcasted_iota(jnp.int32, (E, E), 1)
    tril8 = (e_c < e_r).astype(jnp.float32)
    off = lax.dot_general(tril8, pc.astype(jnp.float32),
                          (((1,), (0,)), ((), ())),
                          preferred_element_type=jnp.float32
                          ).astype(jnp.int32)
    rank_i = rank_f.astype(jnp.int32)

    slot_f = (off + rank_i).astype(jnp.float32)
    cb = jnp.zeros((1, T), jnp.float32)
    slotA = jnp.zeros((1, T), jnp.float32)
    slotB = jnp.zeros((1, T), jnp.float32)
    wA = jnp.zeros((1, T), jnp.float32)
    wB = jnp.zeros((1, T), jnp.float32)
    for e in range(E):
        f = sel_f[e]
        first = f * (cb == 0.0).astype(jnp.float32)
        second = f * (cb == 1.0).astype(jnp.float32)
        row = slot_f[e:e + 1, :]
        slotA = slotA + first * row
        slotB = slotB + second * row
        wA = wA + first * wn[e]
        wB = wB + second * wn[e]
        cb = cb + f
    s2_ref[...] = jnp.concatenate([slotA, slotB], axis=0).astype(jnp.int32)
    zero = jnp.zeros((1, T), jnp.float32)
    w2_t = jnp.concatenate([wA, wB] + [zero] * (E - 2), axis=0)
    w2_ref[...] = w2_t.T

    bidx = lax.broadcasted_iota(jnp.int32, (E, 64), 1)
    boff = off >> LOG_BLK
    ge = (bidx >= boff).astype(jnp.int32)
    cnt = jnp.sum(ge, axis=0, keepdims=True)
    be_row = jnp.clip(cnt - 1, 0, E - 1)
    nb = jnp.sum(pc, axis=0, keepdims=True) >> LOG_BLK
    bi1 = lax.broadcasted_iota(jnp.int32, (1, 64), 1)
    be_row = jnp.where(bi1 == 63, nb, be_row)
    be_ref[...] = be_row + jnp.zeros((E, 64), jnp.int32)

    xp_ref[...] = _pack_bf16(x[:, :HH], x[:, HH:])


def _build_dispatch():
    mesh = plsc.VectorSubcoreMesh(core_axis_name="c", subcore_axis_name="s")
    rows_per = (2 * T) // NTILES

    @functools.partial(
        pl.kernel,
        mesh=mesh,
        out_type=jax.ShapeDtypeStruct((NSLOT, HH), jnp.int32),
        scratch_types=[
            pltpu.VMEM((2, RCH // 2), jnp.int32),
            pltpu.VMEM((RCH, HH), jnp.int32),
            pltpu.SemaphoreType.DMA,
            pltpu.SemaphoreType.DMA,
        ],
    )
    def _dispatch(s2_hbm, x_hbm, disp_hbm, idx_v, rows_v, ld_sem, st_sem):
        c = lax.axis_index("c")
        s = lax.axis_index("s")
        wid = s * 2 + c
        flat0 = pl.multiple_of(wid * rows_per, rows_per)
        t0 = pl.multiple_of(flat0 & (T - 1), rows_per)
        hch = RCH // 2
        pltpu.sync_copy(s2_hbm.at[pl.ds(wid * 2, 2)], idx_v)
        loads = [pltpu.async_copy(
            x_hbm.at[pl.ds(t0 + j * hch, hch)],
            rows_v.at[pl.ds(j * hch, hch)], ld_sem) for j in range(2)]
        stores = []
        for j in range(2):
            loads[j].wait()
            stores.append(pltpu.async_copy(
                rows_v.at[pl.ds(j * hch, hch)],
                disp_hbm.at[idx_v.at[j]], st_sem))
        for st in stores:
            st.wait()

    return _dispatch


def _build_gather():
    mesh = plsc.VectorSubcoreMesh(core_axis_name="c", subcore_axis_name="s")
    rows_per = (2 * T) // NTILES

    @functools.partial(
        pl.kernel,
        mesh=mesh,
        out_type=jax.ShapeDtypeStruct((2 * T, HH), jnp.int32),
        scratch_types=[
            pltpu.VMEM((2, RCH // 2), jnp.int32),
            pltpu.VMEM((RCH, HH), jnp.int32),
            pltpu.SemaphoreType.DMA,
            pltpu.SemaphoreType.DMA,
        ],
    )
    def _gather(idx_hbm, tab_hbm, out_hbm, idx_v, rows_v, g_sem, st_sem):
        c = lax.axis_index("c")
        s = lax.axis_index("s")
        wid = s * 2 + c
        base = pl.multiple_of(wid * rows_per, rows_per)
        hch = RCH // 2
        pltpu.sync_copy(idx_hbm.at[pl.ds(wid * 2, 2)], idx_v)
        gathers = [pltpu.async_copy(
            tab_hbm.at[idx_v.at[j]],
            rows_v.at[pl.ds(j * hch, hch)], g_sem) for j in range(2)]
        stores = []
        for j in range(2):
            gathers[j].wait()
            stores.append(pltpu.async_copy(
                rows_v.at[pl.ds(j * hch, hch)],
                out_hbm.at[pl.ds(base + j * hch, hch)], st_sem))
        for st in stores:
            st.wait()

    return _gather


@functools.lru_cache(maxsize=None)
def _sc_kernels():
    return (_build_dispatch(), _build_gather())


def _sc_dispatch(s2_chunked, xp):
    return _sc_kernels()[0](s2_chunked, xp)


def _s3_gather(s2_chunked, yp):
    return _sc_kernels()[1](s2_chunked, yp)


def _ffn_sparse_kernel(be_ref, disp_ref, wg_ref, wu_ref, wd_ref, y_ref):
    b = pl.program_id(0)
    nb = be_ref[63]

    @pl.when(b < nb)
    def _():
        xlo_f, xhi_f = _unpack_bf16(disp_ref[...])
        xlo = xlo_f.astype(jnp.bfloat16)
        xhi = xhi_f.astype(jnp.bfloat16)
        wg = wg_ref[0]
        wu = wu_ref[0]
        hg = (lax.dot_general(xlo, wg[:, :HH].astype(jnp.bfloat16),
                              (((1,), (1,)), ((), ())),
                              preferred_element_type=jnp.float32) +
              lax.dot_general(xhi, wg[:, HH:].astype(jnp.bfloat16),
                              (((1,), (1,)), ((), ())),
                              preferred_element_type=jnp.float32))
        hu = (lax.dot_general(xlo, wu[:, :HH].astype(jnp.bfloat16),
                              (((1,), (1,)), ((), ())),
                              preferred_element_type=jnp.float32) +
              lax.dot_general(xhi, wu[:, HH:].astype(jnp.bfloat16),
                              (((1,), (1,)), ((), ())),
                              preferred_element_type=jnp.float32))
        inter = (hg * jax.nn.sigmoid(hg)) * hu
        y = lax.dot_general(inter.astype(jnp.bfloat16),
                            wd_ref[0].astype(jnp.bfloat16),
                            (((1,), (1,)), ((), ())),
                            preferred_element_type=jnp.float32)
        y_ref[...] = _pack_bf16(y[:, :HH], y[:, HH:])


def _final_kernel(xp_ref, wsg_ref, wsu_ref, wsd_ref, y2_ref, w2_ref, out_ref):
    xlo_f, xhi_f = _unpack_bf16(xp_ref[...])
    xlo = xlo_f.astype(jnp.bfloat16)
    xhi = xhi_f.astype(jnp.bfloat16)
    wsg = wsg_ref[...]
    wsu = wsu_ref[...]
    hg = (lax.dot_general(xlo, wsg[:, :HH].astype(jnp.bfloat16),
                          (((1,), (1,)), ((), ())),
                          preferred_element_type=jnp.float32) +
          lax.dot_general(xhi, wsg[:, HH:].astype(jnp.bfloat16),
                          (((1,), (1,)), ((), ())),
                          preferred_element_type=jnp.float32))
    hu = (lax.dot_general(xlo, wsu[:, :HH].astype(jnp.bfloat16),
                          (((1,), (1,)), ((), ())),
                          preferred_element_type=jnp.float32) +
          lax.dot_general(xhi, wsu[:, HH:].astype(jnp.bfloat16),
                          (((1,), (1,)), ((), ())),
                          preferred_element_type=jnp.float32))
    inter = (hg * jax.nn.sigmoid(hg)) * hu
    sh = lax.dot_general(inter.astype(jnp.bfloat16),
                         wsd_ref[...].astype(jnp.bfloat16),
                         (((1,), (1,)), ((), ())),
                         preferred_element_type=jnp.float32)
    loA, hiA = _unpack_bf16(y2_ref[0])
    loB, hiB = _unpack_bf16(y2_ref[1])
    wA = w2_ref[:, 0:1]
    wB = w2_ref[:, 1:2]
    out_ref[:, :HH] = sh[:, :HH] + loA * wA + loB * wB
    out_ref[:, HH:] = sh[:, HH:] + hiA * wA + hiB * wB


def kernel(hidden_states, gate_w, e_bias, gate_proj, up_proj, down_proj,
           shared_gate_w, shared_up_w, shared_down_w):
    b, ss, h = hidden_states.shape
    x = hidden_states.reshape(T, H)

    w2, s2, be_out, xp = pl.pallas_call(
        _routing_kernel,
        out_shape=(
            jax.ShapeDtypeStruct((T, E), jnp.float32),
            jax.ShapeDtypeStruct((2, T), jnp.int32),
            jax.ShapeDtypeStruct((E, 64), jnp.int32),
            jax.ShapeDtypeStruct((T, HH), jnp.int32),
        ),
    )(x, gate_w, e_bias.reshape(E, 1))

    s2_chunked = s2.reshape(2 * T // (RCH // 2), RCH // 2)
    disp = _sc_dispatch(s2_chunked, xp)

    be_arr = be_out[0]

    grid_spec = pltpu.PrefetchScalarGridSpec(
        num_scalar_prefetch=1,
        grid=(NBLK,),
        in_specs=[
            pl.BlockSpec((BLK, HH), lambda bb, be: (bb, 0)),
            pl.BlockSpec((1, I, H), lambda bb, be: (be[bb], 0, 0)),
            pl.BlockSpec((1, I, H), lambda bb, be: (be[bb], 0, 0)),
            pl.BlockSpec((1, H, I), lambda bb, be: (be[bb], 0, 0)),
        ],
        out_specs=pl.BlockSpec((BLK, HH), lambda bb, be: (bb, 0)),
    )
    yp = pl.pallas_call(
        _ffn_sparse_kernel,
        grid_spec=grid_spec,
        out_shape=jax.ShapeDtypeStruct((NSLOT, HH), jnp.int32),
    )(be_arr, disp, gate_proj, up_proj, down_proj)

    y2 = _s3_gather(s2_chunked, yp).reshape(2, T, HH)

    out = pl.pallas_call(
        _final_kernel,
        out_shape=jax.ShapeDtypeStruct((T, H), jnp.float32),
    )(xp, shared_gate_w, shared_up_w, shared_down_w, y2, w2)

    return out.reshape(b, ss, h)


# speedup vs baseline: 1.1995x; 1.1995x over previous
import functools

import jax
import jax.numpy as jnp
from jax import lax
from jax.experimental import pallas as pl
from jax.experimental.pallas import tpu as pltpu
from jax.experimental.pallas import tpu_sc as plsc

H = 1024
HH = H // 2
I = 512
E = 8
NG = 4
T = 2048
BLK = 256
LOG_BLK = 8
NBLK = 24
NSLOT = NBLK * BLK
NTILES = 32
RCH = 128


def _pack_bf16(lo_f32, hi_f32):
    lo = lax.bitcast_convert_type(lo_f32.astype(jnp.bfloat16),
                                  jnp.uint16).astype(jnp.uint32)
    hi = lax.bitcast_convert_type(hi_f32.astype(jnp.bfloat16),
                                  jnp.uint16).astype(jnp.uint32)
    return lax.bitcast_convert_type(lo | (hi << 16), jnp.int32)


def _unpack_bf16(p):
    lo = lax.bitcast_convert_type(p << 16, jnp.float32)
    hi = lax.bitcast_convert_type(p & jnp.int32(-65536), jnp.float32)
    return lo, hi


def _routing_kernel(x_ref, gw_ref, eb_ref, w2_ref, s2_ref, be_ref, xp_ref):
    x = x_ref[...]
    gw = gw_ref[...]
    logits_t = lax.dot_general(gw, x, (((1,), (1,)), ((), ())),
                               preferred_element_type=jnp.float32)
    scores = jax.nn.sigmoid(logits_t)
    sfc = scores + eb_ref[...]
    s = [sfc[e:e + 1, :] for e in range(E)]
    sc = [scores[e:e + 1, :] for e in range(E)]
    g = [s[2 * i] + s[2 * i + 1] for i in range(NG)]
    gsel = []
    for i in range(NG):
        r = jnp.zeros_like(g[0])
        for j in range(NG):
            if j == i:
                continue
            beats = (g[j] > g[i]) | ((g[j] == g[i]) & (j < i))
            r = r + beats.astype(jnp.float32)
        gsel.append(r < 2.0)
    tmp = [jnp.where(gsel[e // 2], s[e], 0.0) for e in range(E)]
    esel = []
    for e in range(E):
        r = jnp.zeros_like(g[0])
        for e2 in range(E):
            if e2 == e:
                continue
            beats = (tmp[e2] > tmp[e]) | ((tmp[e2] == tmp[e]) & (e2 < e))
            r = r + beats.astype(jnp.float32)
        esel.append(r < 2.0)
    w = [jnp.where(esel[e], sc[e], 0.0) for e in range(E)]
    wsum = w[0]
    for e in range(1, E):
        wsum = wsum + w[e]
    inv = 1.0 / (wsum + 1e-20)
    wn = [w[e] * inv for e in range(E)]
    sel_f = [esel[e].astype(jnp.float32) for e in range(E)]
    sel_mat = jnp.concatenate(sel_f, axis=0)

    CH = 256
    iot_r = lax.broadcasted_iota(jnp.int32, (CH, CH), 0)
    iot_c = lax.broadcasted_iota(jnp.int32, (CH, CH), 1)
    tril = (iot_r < iot_c).astype(jnp.float32)
    ones_ch = jnp.ones((CH, 1), jnp.float32)
    carry = jnp.zeros((E, 1), jnp.float32)
    rank_chunks = []
    for cidx in range(T // CH):
        blk = sel_mat[:, cidx * CH:(cidx + 1) * CH]
        rank_chunks.append(
            lax.dot_general(blk, tril, (((1,), (0,)), ((), ())),
                            preferred_element_type=jnp.float32) + carry)
        carry = carry + lax.dot_general(blk, ones_ch, (((1,), (0,)), ((), ())),
                                        preferred_element_type=jnp.float32)
    rank_f = jnp.concatenate(rank_chunks, axis=1)
    counts = carry
    c_i = counts.astype(jnp.int32)
    pc = ((c_i + BLK - 1) >> LOG_BLK) << LOG_BLK
    e_r = lax.broadcasted_iota(jnp.int32, (E, E), 0)
    e_c = lax.broadcasted_iota(jnp.int32, (E, E), 1)
    tril8 = (e_c < e_r).astype(jnp.float32)
    off = lax.dot_general(tril8, pc.astype(jnp.float32),
                          (((1,), (0,)), ((), ())),
                          preferred_element_type=jnp.float32
                          ).astype(jnp.int32)
    rank_i = rank_f.astype(jnp.int32)

    slot_f = (off + rank_i).astype(jnp.float32)
    cb = jnp.zeros((1, T), jnp.float32)
    slotA = jnp.zeros((1, T), jnp.float32)
    slotB = jnp.zeros((1, T), jnp.float32)
    wA = jnp.zeros((1, T), jnp.float32)
    wB = jnp.zeros((1, T), jnp.float32)
    for e in range(E):
        f = sel_f[e]
        first = f * (cb == 0.0).astype(jnp.float32)
        second = f * (cb == 1.0).astype(jnp.float32)
        row = slot_f[e:e + 1, :]
        slotA = slotA + first * row
        slotB = slotB + second * row
        wA = wA + first * wn[e]
        wB = wB + second * wn[e]
        cb = cb + f
    s2_ref[...] = jnp.concatenate([slotA, slotB], axis=0).astype(jnp.int32)
    zero = jnp.zeros((1, T), jnp.float32)
    w2_t = jnp.concatenate([wA, wB] + [zero] * (E - 2), axis=0)
    w2_ref[...] = w2_t.T

    bidx = lax.broadcasted_iota(jnp.int32, (E, 64), 1)
    boff = off >> LOG_BLK
    ge = (bidx >= boff).astype(jnp.int32)
    cnt = jnp.sum(ge, axis=0, keepdims=True)
    be_row = jnp.clip(cnt - 1, 0, E - 1)
    nb = jnp.sum(pc, axis=0, keepdims=True) >> LOG_BLK
    bi1 = lax.broadcasted_iota(jnp.int32, (1, 64), 1)
    be_row = jnp.where(bi1 == 63, nb, be_row)
    be_ref[...] = be_row + jnp.zeros((E, 64), jnp.int32)

    xp_ref[...] = _pack_bf16(x[:, :HH], x[:, HH:])


def _build_dispatch():
    mesh = plsc.VectorSubcoreMesh(core_axis_name="c", subcore_axis_name="s")
    rows_per = (2 * T) // NTILES

    @functools.partial(
        pl.kernel,
        mesh=mesh,
        out_type=jax.ShapeDtypeStruct((NSLOT, HH), jnp.int32),
        scratch_types=[
            pltpu.VMEM((2, RCH // 2), jnp.int32),
            pltpu.VMEM((RCH, HH), jnp.int32),
            pltpu.SemaphoreType.DMA,
            pltpu.SemaphoreType.DMA,
        ],
    )
    def _dispatch(s2_hbm, x_hbm, disp_hbm, idx_v, rows_v, ld_sem, st_sem):
        c = lax.axis_index("c")
        s = lax.axis_index("s")
        wid = s * 2 + c
        flat0 = pl.multiple_of(wid * rows_per, rows_per)
        t0 = pl.multiple_of(flat0 & (T - 1), rows_per)
        hch = RCH // 2
        pltpu.sync_copy(s2_hbm.at[pl.ds(wid * 2, 2)], idx_v)
        loads = [pltpu.async_copy(
            x_hbm.at[pl.ds(t0 + j * hch, hch)],
            rows_v.at[pl.ds(j * hch, hch)], ld_sem) for j in range(2)]
        stores = []
        for j in range(2):
            loads[j].wait()
            stores.append(pltpu.async_copy(
                rows_v.at[pl.ds(j * hch, hch)],
                disp_hbm.at[idx_v.at[j]], st_sem))
        for st in stores:
            st.wait()

    return _dispatch


def _build_gather():
    mesh = plsc.VectorSubcoreMesh(core_axis_name="c", subcore_axis_name="s")
    rows_per = (2 * T) // NTILES

    @functools.partial(
        pl.kernel,
        mesh=mesh,
        out_type=jax.ShapeDtypeStruct((2 * T, HH), jnp.int32),
        scratch_types=[
            pltpu.VMEM((2, RCH // 2), jnp.int32),
            pltpu.VMEM((RCH, HH), jnp.int32),
            pltpu.SemaphoreType.DMA,
            pltpu.SemaphoreType.DMA,
        ],
    )
    def _gather(idx_hbm, tab_hbm, out_hbm, idx_v, rows_v, g_sem, st_sem):
        c = lax.axis_index("c")
        s = lax.axis_index("s")
        wid = s * 2 + c
        base = pl.multiple_of(wid * rows_per, rows_per)
        hch = RCH // 2
        pltpu.sync_copy(idx_hbm.at[pl.ds(wid * 2, 2)], idx_v)
        gathers = [pltpu.async_copy(
            tab_hbm.at[idx_v.at[j]],
            rows_v.at[pl.ds(j * hch, hch)], g_sem) for j in range(2)]
        stores = []
        for j in range(2):
            gathers[j].wait()
            stores.append(pltpu.async_copy(
                rows_v.at[pl.ds(j * hch, hch)],
                out_hbm.at[pl.ds(base + j * hch, hch)], st_sem))
        for st in stores:
            st.wait()

    return _gather


@functools.lru_cache(maxsize=None)
def _sc_kernels():
    return (_build_dispatch(), _build_gather())


def _sc_dispatch(s2_chunked, xp):
    return _sc_kernels()[0](s2_chunked, xp)


def _s3_gather(s2_chunked, yp):
    return _sc_kernels()[1](s2_chunked, yp)


def _ffn_sparse_kernel(be_ref, disp_ref, wg_ref, wu_ref, wd_ref, y_ref):
    b = pl.program_id(0)
    nb = be_ref[63]

    @pl.when(b < nb)
    def _():
        xlo_f, xhi_f = _unpack_bf16(disp_ref[...])
        xlo = xlo_f.astype(jnp.bfloat16)
        xhi = xhi_f.astype(jnp.bfloat16)
        wg = wg_ref[0]
        wu = wu_ref[0]
        hg = (lax.dot_general(xlo, wg[:, :HH].astype(jnp.bfloat16),
                              (((1,), (1,)), ((), ())),
                              preferred_element_type=jnp.float32) +
              lax.dot_general(xhi, wg[:, HH:].astype(jnp.bfloat16),
                              (((1,), (1,)), ((), ())),
                              preferred_element_type=jnp.float32))
        hu = (lax.dot_general(xlo, wu[:, :HH].astype(jnp.bfloat16),
                              (((1,), (1,)), ((), ())),
                              preferred_element_type=jnp.float32) +
              lax.dot_general(xhi, wu[:, HH:].astype(jnp.bfloat16),
                              (((1,), (1,)), ((), ())),
                              preferred_element_type=jnp.float32))
        inter = (hg * jax.nn.sigmoid(hg)) * hu
        y = lax.dot_general(inter.astype(jnp.bfloat16),
                            wd_ref[0].astype(jnp.bfloat16),
                            (((1,), (1,)), ((), ())),
                            preferred_element_type=jnp.float32)
        y_ref[...] = _pack_bf16(y[:, :HH], y[:, HH:])


def _final_kernel(xp_ref, wsg_ref, wsu_ref, wsd_ref, y2_ref, w2_ref, out_ref):
    xlo_f, xhi_f = _unpack_bf16(xp_ref[...])
    xlo = xlo_f.astype(jnp.bfloat16)
    xhi = xhi_f.astype(jnp.bfloat16)
    wsg = wsg_ref[...]
    wsu = wsu_ref[...]
    hg = (lax.dot_general(xlo, wsg[:, :HH].astype(jnp.bfloat16),
                          (((1,), (1,)), ((), ())),
                          preferred_element_type=jnp.float32) +
          lax.dot_general(xhi, wsg[:, HH:].astype(jnp.bfloat16),
                          (((1,), (1,)), ((), ())),
                          preferred_element_type=jnp.float32))
    hu = (lax.dot_general(xlo, wsu[:, :HH].astype(jnp.bfloat16),
                          (((1,), (1,)), ((), ())),
                          preferred_element_type=jnp.float32) +
          lax.dot_general(xhi, wsu[:, HH:].astype(jnp.bfloat16),
                          (((1,), (1,)), ((), ())),
                          preferred_element_type=jnp.float32))
    inter = (hg * jax.nn.sigmoid(hg)) * hu
    sh = lax.dot_general(inter.astype(jnp.bfloat16),
                         wsd_ref[...].astype(jnp.bfloat16),
                         (((1,), (1,)), ((), ())),
                         preferred_element_type=jnp.float32)
    loA, hiA = _unpack_bf16(y2_ref[0])
    loB, hiB = _unpack_bf16(y2_ref[1])
    wA = w2_ref[:, 0:1]
    wB = w2_ref[:, 1:2]
    out_ref[:, :HH] = sh[:, :HH] + loA * wA + loB * wB
    out_ref[:, HH:] = sh[:, HH:] + hiA * wA + hiB * wB


def kernel(hidden_states, gate_w, e_bias, gate_proj, up_proj, down_proj,
           shared_gate_w, shared_up_w, shared_down_w):
    b, ss, h = hidden_states.shape
    x = hidden_states.reshape(T, H)

    w2, s2, be_out, xp = pl.pallas_call(
        _routing_kernel,
        out_shape=(
            jax.ShapeDtypeStruct((T, E), jnp.float32),
            jax.ShapeDtypeStruct((2, T), jnp.int32),
            jax.ShapeDtypeStruct((E, 64), jnp.int32),
            jax.ShapeDtypeStruct((T, HH), jnp.int32),
        ),
    )(x, gate_w, e_bias.reshape(E, 1))

    s2_chunked = s2.reshape(2 * T // (RCH // 2), RCH // 2)
    disp = _sc_dispatch(s2_chunked, xp)

    be_arr = be_out[0]

    grid_spec = pltpu.PrefetchScalarGridSpec(
        num_scalar_prefetch=1,
        grid=(NBLK,),
        in_specs=[
            pl.BlockSpec((BLK, HH), lambda bb, be: (bb, 0)),
            pl.BlockSpec((1, I, H), lambda bb, be: (be[bb], 0, 0)),
            pl.BlockSpec((1, I, H), lambda bb, be: (be[bb], 0, 0)),
            pl.BlockSpec((1, H, I), lambda bb, be: (be[bb], 0, 0)),
        ],
        out_specs=pl.BlockSpec((BLK, HH), lambda bb, be: (bb, 0)),
    )
    yp = pl.pallas_call(
        _ffn_sparse_kernel,
        grid_spec=grid_spec,
        out_shape=jax.ShapeDtypeStruct((NSLOT, HH), jnp.int32),
    )(be_arr, disp, gate_proj, up_proj, down_proj)

    y2 = _s3_gather(s2_chunked, yp).reshape(2, T, HH)

    out = pl.pallas_call(
        _final_kernel,
        out_shape=jax.ShapeDtypeStruct((T, H), jnp.float32),
    )(xp, shared_gate_w, shared_up_w, shared_down_w, y2, w2)

    return out.reshape(b, ss, h)


# speedup vs baseline: 1.3067x; 1.0893x over previous
import functools

import jax
import jax.numpy as jnp
from jax import lax
from jax.experimental import pallas as pl
from jax.experimental.pallas import tpu as pltpu
from jax.experimental.pallas import tpu_sc as plsc

H = 1024
HH = H // 2
I = 512
E = 8
NG = 4
T = 2048
BLK = 512
LOG_BLK = 9
NBLK = 16
NSLOT = NBLK * BLK
NTILES = 32
RCH = 128


def _pack_bf16(lo_f32, hi_f32):
    lo = lax.bitcast_convert_type(lo_f32.astype(jnp.bfloat16),
                                  jnp.uint16).astype(jnp.uint32)
    hi = lax.bitcast_convert_type(hi_f32.astype(jnp.bfloat16),
                                  jnp.uint16).astype(jnp.uint32)
    return lax.bitcast_convert_type(lo | (hi << 16), jnp.int32)


def _unpack_bf16(p):
    lo = lax.bitcast_convert_type(p << 16, jnp.float32)
    hi = lax.bitcast_convert_type(p & jnp.int32(-65536), jnp.float32)
    return lo, hi


def _routing_kernel(x_ref, gw_ref, eb_ref, w2_ref, s2_ref, be_ref, xp_ref):
    x = x_ref[...]
    gw = gw_ref[...]
    logits_t = lax.dot_general(gw, x, (((1,), (1,)), ((), ())),
                               preferred_element_type=jnp.float32)
    scores = jax.nn.sigmoid(logits_t)
    sfc = scores + eb_ref[...]
    s = [sfc[e:e + 1, :] for e in range(E)]
    sc = [scores[e:e + 1, :] for e in range(E)]
    g = [s[2 * i] + s[2 * i + 1] for i in range(NG)]
    gsel = []
    for i in range(NG):
        r = jnp.zeros_like(g[0])
        for j in range(NG):
            if j == i:
                continue
            beats = (g[j] > g[i]) | ((g[j] == g[i]) & (j < i))
            r = r + beats.astype(jnp.float32)
        gsel.append(r < 2.0)
    tmp = [jnp.where(gsel[e // 2], s[e], 0.0) for e in range(E)]
    esel = []
    for e in range(E):
        r = jnp.zeros_like(g[0])
        for e2 in range(E):
            if e2 == e:
                continue
            beats = (tmp[e2] > tmp[e]) | ((tmp[e2] == tmp[e]) & (e2 < e))
            r = r + beats.astype(jnp.float32)
        esel.append(r < 2.0)
    w = [jnp.where(esel[e], sc[e], 0.0) for e in range(E)]
    wsum = w[0]
    for e in range(1, E):
        wsum = wsum + w[e]
    inv = 1.0 / (wsum + 1e-20)
    wn = [w[e] * inv for e in range(E)]
    sel_f = [esel[e].astype(jnp.float32) for e in range(E)]
    sel_mat = jnp.concatenate(sel_f, axis=0)

    CH = 256
    iot_r = lax.broadcasted_iota(jnp.int32, (CH, CH), 0)
    iot_c = lax.broadcasted_iota(jnp.int32, (CH, CH), 1)
    tril = (iot_r < iot_c).astype(jnp.float32)
    ones_ch = jnp.ones((CH, 1), jnp.float32)
    carry = jnp.zeros((E, 1), jnp.float32)
    rank_chunks = []
    for cidx in range(T // CH):
        blk = sel_mat[:, cidx * CH:(cidx + 1) * CH]
        rank_chunks.append(
            lax.dot_general(blk, tril, (((1,), (0,)), ((), ())),
                            preferred_element_type=jnp.float32) + carry)
        carry = carry + lax.dot_general(blk, ones_ch, (((1,), (0,)), ((), ())),
                                        preferred_element_type=jnp.float32)
    rank_f = jnp.concatenate(rank_chunks, axis=1)
    counts = carry
    c_i = counts.astype(jnp.int32)
    pc = ((c_i + BLK - 1) >> LOG_BLK) << LOG_BLK
    e_r = lax.broadcasted_iota(jnp.int32, (E, E), 0)
    e_c = lax.broadcasted_iota(jnp.int32, (E, E), 1)
    tril8 = (e_c < e_r).astype(jnp.float32)
    off = lax.dot_general(tril8, pc.astype(jnp.float32),
                          (((1,), (0,)), ((), ())),
                          preferred_element_type=jnp.float32
                          ).astype(jnp.int32)
    rank_i = rank_f.astype(jnp.int32)

    slot_f = (off + rank_i).astype(jnp.float32)
    cb = jnp.zeros((1, T), jnp.float32)
    slotA = jnp.zeros((1, T), jnp.float32)
    slotB = jnp.zeros((1, T), jnp.float32)
    wA = jnp.zeros((1, T), jnp.float32)
    wB = jnp.zeros((1, T), jnp.float32)
    for e in range(E):
        f = sel_f[e]
        first = f * (cb == 0.0).astype(jnp.float32)
        second = f * (cb == 1.0).astype(jnp.float32)
        row = slot_f[e:e + 1, :]
        slotA = slotA + first * row
        slotB = slotB + second * row
        wA = wA + first * wn[e]
        wB = wB + second * wn[e]
        cb = cb + f
    s2_ref[...] = jnp.concatenate([slotA, slotB], axis=0).astype(jnp.int32)
    zero = jnp.zeros((1, T), jnp.float32)
    w2_t = jnp.concatenate([wA, wB] + [zero] * (E - 2), axis=0)
    w2_ref[...] = w2_t.T

    bidx = lax.broadcasted_iota(jnp.int32, (E, 64), 1)
    boff = off >> LOG_BLK
    ge = (bidx >= boff).astype(jnp.int32)
    cnt = jnp.sum(ge, axis=0, keepdims=True)
    be_row = jnp.clip(cnt - 1, 0, E - 1)
    nb = jnp.sum(pc, axis=0, keepdims=True) >> LOG_BLK
    bi1 = lax.broadcasted_iota(jnp.int32, (1, 64), 1)
    be_row = jnp.where(bi1 == 63, nb, be_row)
    be_ref[...] = be_row + jnp.zeros((E, 64), jnp.int32)

    xp_ref[...] = _pack_bf16(x[:, :HH], x[:, HH:])


def _build_dispatch():
    mesh = plsc.VectorSubcoreMesh(core_axis_name="c", subcore_axis_name="s")
    rows_per = (2 * T) // NTILES

    @functools.partial(
        pl.kernel,
        mesh=mesh,
        out_type=jax.ShapeDtypeStruct((NSLOT, HH), jnp.int32),
        scratch_types=[
            pltpu.VMEM((2, RCH // 2), jnp.int32),
            pltpu.VMEM((RCH, HH), jnp.int32),
            pltpu.SemaphoreType.DMA,
            pltpu.SemaphoreType.DMA,
        ],
    )
    def _dispatch(s2_hbm, x_hbm, disp_hbm, idx_v, rows_v, ld_sem, st_sem):
        c = lax.axis_index("c")
        s = lax.axis_index("s")
        wid = s * 2 + c
        flat0 = pl.multiple_of(wid * rows_per, rows_per)
        t0 = pl.multiple_of(flat0 & (T - 1), rows_per)
        hch = RCH // 2
        pltpu.sync_copy(s2_hbm.at[pl.ds(wid * 2, 2)], idx_v)
        loads = [pltpu.async_copy(
            x_hbm.at[pl.ds(t0 + j * hch, hch)],
            rows_v.at[pl.ds(j * hch, hch)], ld_sem) for j in range(2)]
        stores = []
        for j in range(2):
            loads[j].wait()
            stores.append(pltpu.async_copy(
                rows_v.at[pl.ds(j * hch, hch)],
                disp_hbm.at[idx_v.at[j]], st_sem))
        for st in stores:
            st.wait()

    return _dispatch


def _build_gather():
    mesh = plsc.VectorSubcoreMesh(core_axis_name="c", subcore_axis_name="s")
    rows_per = (2 * T) // NTILES

    @functools.partial(
        pl.kernel,
        mesh=mesh,
        out_type=jax.ShapeDtypeStruct((2 * T, HH), jnp.int32),
        scratch_types=[
            pltpu.VMEM((2, RCH // 2), jnp.int32),
            pltpu.VMEM((RCH, HH), jnp.int32),
            pltpu.SemaphoreType.DMA,
            pltpu.SemaphoreType.DMA,
        ],
    )
    def _gather(idx_hbm, tab_hbm, out_hbm, idx_v, rows_v, g_sem, st_sem):
        c = lax.axis_index("c")
        s = lax.axis_index("s")
        wid = s * 2 + c
        base = pl.multiple_of(wid * rows_per, rows_per)
        hch = RCH // 2
        pltpu.sync_copy(idx_hbm.at[pl.ds(wid * 2, 2)], idx_v)
        gathers = [pltpu.async_copy(
            tab_hbm.at[idx_v.at[j]],
            rows_v.at[pl.ds(j * hch, hch)], g_sem) for j in range(2)]
        stores = []
        for j in range(2):
            gathers[j].wait()
            stores.append(pltpu.async_copy(
                rows_v.at[pl.ds(j * hch, hch)],
                out_hbm.at[pl.ds(base + j * hch, hch)], st_sem))
        for st in stores:
            st.wait()

    return _gather


@functools.lru_cache(maxsize=None)
def _sc_kernels():
    return (_build_dispatch(), _build_gather())


def _sc_dispatch(s2_chunked, xp):
    return _sc_kernels()[0](s2_chunked, xp)


def _s3_gather(s2_chunked, yp):
    return _sc_kernels()[1](s2_chunked, yp)


def _ffn_sparse_kernel(be_ref, disp_ref, wg_ref, wu_ref, wd_ref, y_ref):
    b = pl.program_id(0)
    nb = be_ref[63]

    @pl.when(b < nb)
    def _():
        xlo_f, xhi_f = _unpack_bf16(disp_ref[...])
        xlo = xlo_f.astype(jnp.bfloat16)
        xhi = xhi_f.astype(jnp.bfloat16)
        wg = wg_ref[0]
        wu = wu_ref[0]
        hg = (lax.dot_general(xlo, wg[:, :HH].astype(jnp.bfloat16),
                              (((1,), (1,)), ((), ())),
                              preferred_element_type=jnp.float32) +
              lax.dot_general(xhi, wg[:, HH:].astype(jnp.bfloat16),
                              (((1,), (1,)), ((), ())),
                              preferred_element_type=jnp.float32))
        hu = (lax.dot_general(xlo, wu[:, :HH].astype(jnp.bfloat16),
                              (((1,), (1,)), ((), ())),
                              preferred_element_type=jnp.float32) +
              lax.dot_general(xhi, wu[:, HH:].astype(jnp.bfloat16),
                              (((1,), (1,)), ((), ())),
                              preferred_element_type=jnp.float32))
        inter = (hg * jax.nn.sigmoid(hg)) * hu
        y = lax.dot_general(inter.astype(jnp.bfloat16),
                            wd_ref[0].astype(jnp.bfloat16),
                            (((1,), (1,)), ((), ())),
                            preferred_element_type=jnp.float32)
        y_ref[...] = _pack_bf16(y[:, :HH], y[:, HH:])


def _final_kernel(xp_ref, wsg_ref, wsu_ref, wsd_ref, y2_ref, w2_ref, out_ref):
    xlo_f, xhi_f = _unpack_bf16(xp_ref[...])
    xlo = xlo_f.astype(jnp.bfloat16)
    xhi = xhi_f.astype(jnp.bfloat16)
    wsg = wsg_ref[...]
    wsu = wsu_ref[...]
    hg = (lax.dot_general(xlo, wsg[:, :HH].astype(jnp.bfloat16),
                          (((1,), (1,)), ((), ())),
                          preferred_element_type=jnp.float32) +
          lax.dot_general(xhi, wsg[:, HH:].astype(jnp.bfloat16),
                          (((1,), (1,)), ((), ())),
                          preferred_element_type=jnp.float32))
    hu = (lax.dot_general(xlo, wsu[:, :HH].astype(jnp.bfloat16),
                          (((1,), (1,)), ((), ())),
                          preferred_element_type=jnp.float32) +
          lax.dot_general(xhi, wsu[:, HH:].astype(jnp.bfloat16),
                          (((1,), (1,)), ((), ())),
                          preferred_element_type=jnp.float32))
    inter = (hg * jax.nn.sigmoid(hg)) * hu
    sh = lax.dot_general(inter.astype(jnp.bfloat16),
                         wsd_ref[...].astype(jnp.bfloat16),
                         (((1,), (1,)), ((), ())),
                         preferred_element_type=jnp.float32)
    loA, hiA = _unpack_bf16(y2_ref[0])
    loB, hiB = _unpack_bf16(y2_ref[1])
    wA = w2_ref[:, 0:1]
    wB = w2_ref[:, 1:2]
    out_ref[:, :HH] = sh[:, :HH] + loA * wA + loB * wB
    out_ref[:, HH:] = sh[:, HH:] + hiA * wA + hiB * wB


def kernel(hidden_states, gate_w, e_bias, gate_proj, up_proj, down_proj,
           shared_gate_w, shared_up_w, shared_down_w):
    b, ss, h = hidden_states.shape
    x = hidden_states.reshape(T, H)

    w2, s2, be_out, xp = pl.pallas_call(
        _routing_kernel,
        out_shape=(
            jax.ShapeDtypeStruct((T, E), jnp.float32),
            jax.ShapeDtypeStruct((2, T), jnp.int32),
            jax.ShapeDtypeStruct((E, 64), jnp.int32),
            jax.ShapeDtypeStruct((T, HH), jnp.int32),
        ),
    )(x, gate_w, e_bias.reshape(E, 1))

    s2_chunked = s2.reshape(2 * T // (RCH // 2), RCH // 2)
    disp = _sc_dispatch(s2_chunked, xp)

    be_arr = be_out[0]

    grid_spec = pltpu.PrefetchScalarGridSpec(
        num_scalar_prefetch=1,
        grid=(NBLK,),
        in_specs=[
            pl.BlockSpec((BLK, HH), lambda bb, be: (bb, 0)),
            pl.BlockSpec((1, I, H), lambda bb, be: (be[bb], 0, 0)),
            pl.BlockSpec((1, I, H), lambda bb, be: (be[bb], 0, 0)),
            pl.BlockSpec((1, H, I), lambda bb, be: (be[bb], 0, 0)),
        ],
        out_specs=pl.BlockSpec((BLK, HH), lambda bb, be: (bb, 0)),
    )
    yp = pl.pallas_call(
        _ffn_sparse_kernel,
        grid_spec=grid_spec,
        out_shape=jax.ShapeDtypeStruct((NSLOT, HH), jnp.int32),
    )(be_arr, disp, gate_proj, up_proj, down_proj)

    y2 = _s3_gather(s2_chunked, yp).reshape(2, T, HH)

    out = pl.pallas_call(
        _final_kernel,
        out_shape=jax.ShapeDtypeStruct((T, H), jnp.float32),
    )(xp, shared_gate_w, shared_up_w, shared_down_w, y2, w2)

    return out.reshape(b, ss, h)
